# baseline (speedup 1.0000x reference)
import functools

import jax
import jax.numpy as jnp
from jax import lax
from jax.experimental import pallas as pl
from jax.experimental.pallas import tpu as pltpu
from jax.experimental.pallas import tpu_sc as plsc

N = 10000
E = 320000
D = 128
NC = 2
NS = 16
NW = NC * NS
EPW = E // NW

CH = 64
NF = EPW // CH
TEDGE = EPW - NF * CH
TOFF = NF * CH

DCH = 128
DNF = EPW // DCH
DTEDGE = EPW - DNF * DCH
DTOFF = DNF * DCH

RPT = 624
TAIL = N - NS * RPT
TBASE = NS * RPT

_mesh = plsc.VectorSubcoreMesh(core_axis_name="c", subcore_axis_name="s")


def _zero_acc(zeros_hbm, acc_sh, s):
    pltpu.sync_copy(zeros_hbm.at[pl.ds(s * RPT, RPT)],
                    acc_sh.at[pl.ds(s * RPT, RPT)])

    @pl.when(s == 0)
    def _():
        pltpu.sync_copy(zeros_hbm.at[pl.ds(TBASE, TAIL)],
                        acc_sh.at[pl.ds(TBASE, TAIL)])


def _write_back(acc_sh, out_hbm, c, s):
    pltpu.sync_copy(acc_sh.at[pl.ds(s * RPT, RPT)],
                    out_hbm.at[c, pl.ds(s * RPT, RPT)])

    @pl.when(s == 0)
    def _():
        pltpu.sync_copy(acc_sh.at[pl.ds(TBASE, TAIL)],
                        out_hbm.at[c, pl.ds(TBASE, TAIL)])


assert DNF == 78


@functools.partial(
    pl.kernel,
    out_type=jax.ShapeDtypeStruct((NC, N, D), jnp.float32),
    mesh=_mesh,
    scratch_types=[
        [pltpu.VMEM((DCH,), jnp.int32) for _ in range(4)],
        pltpu.VMEM((DTEDGE,), jnp.int32),
        pltpu.VMEM((DCH, D), jnp.float32),
        pltpu.VMEM_SHARED((N, D), jnp.float32),
        [pltpu.SemaphoreType.DMA for _ in range(4)],
        [pltpu.SemaphoreType.DMA for _ in range(4)],
        pltpu.SemaphoreType.DMA,
    ],
)
def _deg_kernel(dst_hbm, zeros_hbm, ones_hbm, out_hbm,
                didx, didxt, ones_v, acc_sh, si, ss, sit):
    c = lax.axis_index("c")
    s = lax.axis_index("s")
    wid = s * NC + c
    base = wid * EPW

    def issue_fetch(k, b):
        pltpu.async_copy(dst_hbm.at[pl.ds(base + k * DCH, DCH)],
                         didx[b], si[b])

    def wait_fetch(k, b):
        pltpu.make_async_copy(dst_hbm.at[pl.ds(base + k * DCH, DCH)],
                              didx[b], si[b]).wait()

    def issue_scatter(b):
        pltpu.async_copy(ones_v, acc_sh.at[didx[b]], ss[b], add=True)

    def wait_scatter(b):
        pltpu.make_async_copy(ones_v, acc_sh.at[didx[b]], ss[b]).wait()

    issue_fetch(0, 0)
    issue_fetch(1, 1)
    pltpu.async_copy(dst_hbm.at[pl.ds(base + DTOFF, DTEDGE)], didxt, sit)
    pltpu.sync_copy(ones_hbm, ones_v)
    _zero_acc(zeros_hbm, acc_sh, s)
    plsc.subcore_barrier()

    for v in (0, 1):
        wait_fetch(v, v)
        issue_scatter(v)
        issue_fetch(v + 2, v + 2)
    for v in (2, 3):
        wait_fetch(v, v)
        issue_scatter(v)
        wait_scatter(v - 2)
        issue_fetch(v + 2, v - 2)

    def body(g, carry):
        v0 = 4 * g
        for u in range(4):
            v = v0 + u
            b = u % 4
            b2 = (u + 2) % 4
            wait_fetch(v, b)
            issue_scatter(b)
            wait_scatter(b2)
            issue_fetch(v + 2, b2)
        return carry

    lax.fori_loop(1, 19, body, 0)

    for v in (76, 77):
        wait_fetch(v, v % 4)
        issue_scatter(v % 4)

    pltpu.make_async_copy(dst_hbm.at[pl.ds(base + DTOFF, DTEDGE)],
                          didxt, sit).wait()
    pltpu.sync_copy(ones_v.at[pl.ds(0, DTEDGE)], acc_sh.at[didxt], add=True)

    for b in (2, 3, 0, 1):
        wait_scatter(b)

    plsc.subcore_barrier()
    _write_back(acc_sh, out_hbm, c, s)


assert NF == 156


@functools.partial(
    pl.kernel,
    out_type=jax.ShapeDtypeStruct((NC, N, D), jnp.float32),
    mesh=_mesh,
    scratch_types=[
        pltpu.VMEM((EPW,), jnp.int32),
        [pltpu.VMEM((CH,), jnp.int32) for _ in range(4)],
        pltpu.VMEM((TEDGE,), jnp.int32),
        [pltpu.VMEM((CH, D), jnp.float32) for _ in range(4)],
        pltpu.VMEM((TEDGE, D), jnp.float32),
        pltpu.VMEM_SHARED((N, D), jnp.float32),
        [pltpu.SemaphoreType.DMA for _ in range(4)],
        [pltpu.SemaphoreType.DMA for _ in range(4)],
        [pltpu.SemaphoreType.DMA for _ in range(4)],
        pltpu.SemaphoreType.DMA,
        pltpu.SemaphoreType.DMA,
    ],
)
def _hop_kernel(g_hbm, src_hbm, dst_hbm, zeros_hbm, out_hbm,
                sidx_v, didx, didxt, rows, rowst, acc_sh,
                sg, si, ss, sgt, sit):
    c = lax.axis_index("c")
    s = lax.axis_index("s")
    wid = s * NC + c
    base = wid * EPW

    def issue_fetch(k, b):
        pltpu.async_copy(dst_hbm.at[pl.ds(base + k * CH, CH)], didx[b], si[b])
        pltpu.async_copy(g_hbm.at[sidx_v.at[pl.ds(k * CH, CH)]], rows[b], sg[b])

    def wait_fetch(k, b):
        pltpu.make_async_copy(dst_hbm.at[pl.ds(base + k * CH, CH)],
                              didx[b], si[b]).wait()
        pltpu.make_async_copy(g_hbm.at[sidx_v.at[pl.ds(k * CH, CH)]],
                              rows[b], sg[b]).wait()

    def issue_scatter(b):
        pltpu.async_copy(rows[b], acc_sh.at[didx[b]], ss[b], add=True)

    def wait_scatter(b):
        pltpu.make_async_copy(rows[b], acc_sh.at[didx[b]], ss[b]).wait()

    pltpu.sync_copy(src_hbm.at[pl.ds(base, EPW)], sidx_v)
    issue_fetch(0, 0)
    issue_fetch(1, 1)
    pltpu.async_copy(dst_hbm.at[pl.ds(base + TOFF, TEDGE)], didxt, sit)
    pltpu.async_copy(g_hbm.at[sidx_v.at[pl.ds(TOFF, TEDGE)]], rowst, sgt)

    _zero_acc(zeros_hbm, acc_sh, s)
    plsc.subcore_barrier()

    for v in (0, 1):
        wait_fetch(v, v)
        issue_scatter(v)
        issue_fetch(v + 2, v + 2)
    for v in (2, 3):
        wait_fetch(v, v)
        issue_scatter(v)
        wait_scatter(v - 2)
        issue_fetch(v + 2, v - 2)

    def body(g, carry):
        v0 = 4 * g
        for u in range(4):
            v = v0 + u
            b = u % 4
            b2 = (u + 2) % 4
            wait_fetch(v, b)
            issue_scatter(b)
            wait_scatter(b2)
            issue_fetch(v + 2, b2)
        return carry

    lax.fori_loop(1, 38, body, 0)

    for v in (152, 153):
        b = v % 4
        b2 = (v + 2) % 4
        wait_fetch(v, b)
        issue_scatter(b)
        wait_scatter(b2)
        issue_fetch(v + 2, b2)
    for v in (154, 155):
        b = v % 4
        wait_fetch(v, b)
        issue_scatter(b)

    pltpu.make_async_copy(dst_hbm.at[pl.ds(base + TOFF, TEDGE)],
                          didxt, sit).wait()
    pltpu.make_async_copy(g_hbm.at[sidx_v.at[pl.ds(TOFF, TEDGE)]],
                          rowst, sgt).wait()
    pltpu.sync_copy(rowst, acc_sh.at[didxt], add=True)

    for b in (0, 1, 2, 3):
        wait_scatter(b)

    plsc.subcore_barrier()
    _write_back(acc_sh, out_hbm, c, s)


_BR = 2000


def _norm_body(dacc_ref, feat_ref, g_ref, norm_ref):
    deg = dacc_ref[0, :, 0:1] + dacc_ref[1, :, 0:1]
    deg = jnp.maximum(deg, 1.0)
    nrm = lax.rsqrt(deg)
    norm_ref[...] = nrm
    g_ref[...] = feat_ref[...] * nrm


_norm_call = pl.pallas_call(
    _norm_body,
    grid=(N // _BR,),
    in_specs=[
        pl.BlockSpec((NC, _BR, D), lambda i: (0, i, 0)),
        pl.BlockSpec((_BR, D), lambda i: (i, 0)),
    ],
    out_specs=[
        pl.BlockSpec((_BR, D), lambda i: (i, 0)),
        pl.BlockSpec((_BR, 1), lambda i: (i, 0)),
    ],
    out_shape=[
        jax.ShapeDtypeStruct((N, D), jnp.float32),
        jax.ShapeDtypeStruct((N, 1), jnp.float32),
    ],
)


def _mid_body(p_ref, norm_ref, g_ref):
    nrm = norm_ref[...]
    g_ref[...] = (p_ref[0] + p_ref[1]) * (nrm * nrm)


_mid_call = pl.pallas_call(
    _mid_body,
    grid=(N // _BR,),
    in_specs=[
        pl.BlockSpec((NC, _BR, D), lambda i: (0, i, 0)),
        pl.BlockSpec((_BR, 1), lambda i: (i, 0)),
    ],
    out_specs=pl.BlockSpec((_BR, D), lambda i: (i, 0)),
    out_shape=jax.ShapeDtypeStruct((N, D), jnp.float32),
)


def _fin_body(q_ref, norm_ref, wt_ref, b_ref, out_ref):
    h = (q_ref[0] + q_ref[1]) * norm_ref[...]
    out_ref[...] = (
        jnp.dot(h, wt_ref[...], preferred_element_type=jnp.float32)
        + b_ref[...]
    )


_fin_call = pl.pallas_call(
    _fin_body,
    grid=(N // _BR,),
    in_specs=[
        pl.BlockSpec((NC, _BR, D), lambda i: (0, i, 0)),
        pl.BlockSpec((_BR, 1), lambda i: (i, 0)),
        pl.BlockSpec((D, D), lambda i: (0, 0)),
        pl.BlockSpec((1, D), lambda i: (0, 0)),
    ],
    out_specs=pl.BlockSpec((_BR, D), lambda i: (i, 0)),
    out_shape=jax.ShapeDtypeStruct((N, D), jnp.float32),
)


def kernel(feat, edge_index, W, b):
    ei = edge_index.astype(jnp.int32)
    src = ei[0]
    dst = ei[1]
    zeros = jnp.zeros((N, D), jnp.float32)
    ones = jnp.ones((DCH, D), jnp.float32)

    dacc = _deg_kernel(dst, zeros, ones)
    g1, norm = _norm_call(dacc, feat)
    p = _hop_kernel(g1, src, dst, zeros)
    g2 = _mid_call(p, norm)
    q = _hop_kernel(g2, src, dst, zeros)
    out = _fin_call(q, norm, W.T.astype(jnp.float32), b.reshape(1, D))
    return out


# speedup vs baseline: 1.1473x; 1.1473x over previous
import functools

import jax
import jax.numpy as jnp
from jax import lax
from jax.experimental import pallas as pl
from jax.experimental.pallas import tpu as pltpu
from jax.experimental.pallas import tpu_sc as plsc

N = 10000
E = 320000
D = 128
NC = 2
NS = 16
NW = NC * NS
EPW = E // NW

CH = 80
NF = EPW // CH
TEDGE = 0
TOFF = NF * CH

DCH = 128
DNF = EPW // DCH
DTEDGE = EPW - DNF * DCH
DTOFF = DNF * DCH

RPT = 624
TAIL = N - NS * RPT
TBASE = NS * RPT

_mesh = plsc.VectorSubcoreMesh(core_axis_name="c", subcore_axis_name="s")


def _zero_acc(zeros_hbm, acc_sh, s):
    pltpu.sync_copy(zeros_hbm.at[pl.ds(s * RPT, RPT)],
                    acc_sh.at[pl.ds(s * RPT, RPT)])

    @pl.when(s == 0)
    def _():
        pltpu.sync_copy(zeros_hbm.at[pl.ds(TBASE, TAIL)],
                        acc_sh.at[pl.ds(TBASE, TAIL)])


def _write_back(acc_sh, out_hbm, c, s):
    pltpu.sync_copy(acc_sh.at[pl.ds(s * RPT, RPT)],
                    out_hbm.at[c, pl.ds(s * RPT, RPT)])

    @pl.when(s == 0)
    def _():
        pltpu.sync_copy(acc_sh.at[pl.ds(TBASE, TAIL)],
                        out_hbm.at[c, pl.ds(TBASE, TAIL)])


assert DNF == 78


@functools.partial(
    pl.kernel,
    out_type=jax.ShapeDtypeStruct((NC, N, D), jnp.float32),
    mesh=_mesh,
    scratch_types=[
        [pltpu.VMEM((DCH,), jnp.int32) for _ in range(4)],
        pltpu.VMEM((DTEDGE,), jnp.int32),
        pltpu.VMEM((DCH, D), jnp.float32),
        pltpu.VMEM_SHARED((N, D), jnp.float32),
        [pltpu.SemaphoreType.DMA for _ in range(4)],
        [pltpu.SemaphoreType.DMA for _ in range(4)],
        pltpu.SemaphoreType.DMA,
    ],
)
def _deg_kernel(dst_hbm, zeros_hbm, ones_hbm, out_hbm,
                didx, didxt, ones_v, acc_sh, si, ss, sit):
    c = lax.axis_index("c")
    s = lax.axis_index("s")
    wid = s * NC + c
    base = wid * EPW

    def issue_fetch(k, b):
        pltpu.async_copy(dst_hbm.at[pl.ds(base + k * DCH, DCH)],
                         didx[b], si[b])

    def wait_fetch(k, b):
        pltpu.make_async_copy(dst_hbm.at[pl.ds(base + k * DCH, DCH)],
                              didx[b], si[b]).wait()

    def issue_scatter(b):
        pltpu.async_copy(ones_v, acc_sh.at[didx[b]], ss[b], add=True)

    def wait_scatter(b):
        pltpu.make_async_copy(ones_v, acc_sh.at[didx[b]], ss[b]).wait()

    issue_fetch(0, 0)
    issue_fetch(1, 1)
    pltpu.async_copy(dst_hbm.at[pl.ds(base + DTOFF, DTEDGE)], didxt, sit)
    pltpu.sync_copy(ones_hbm, ones_v)
    _zero_acc(zeros_hbm, acc_sh, s)
    plsc.subcore_barrier()

    for v in (0, 1):
        wait_fetch(v, v)
        issue_scatter(v)
        issue_fetch(v + 2, v + 2)
    for v in (2, 3):
        wait_fetch(v, v)
        issue_scatter(v)
        wait_scatter(v - 2)
        issue_fetch(v + 2, v - 2)

    def body(g, carry):
        v0 = 4 * g
        for u in range(4):
            v = v0 + u
            b = u % 4
            b2 = (u + 2) % 4
            wait_fetch(v, b)
            issue_scatter(b)
            wait_scatter(b2)
            issue_fetch(v + 2, b2)
        return carry

    lax.fori_loop(1, 19, body, 0)

    for v in (76, 77):
        wait_fetch(v, v % 4)
        issue_scatter(v % 4)

    pltpu.make_async_copy(dst_hbm.at[pl.ds(base + DTOFF, DTEDGE)],
                          didxt, sit).wait()
    pltpu.sync_copy(ones_v.at[pl.ds(0, DTEDGE)], acc_sh.at[didxt], add=True)

    for b in (2, 3, 0, 1):
        wait_scatter(b)

    plsc.subcore_barrier()
    _write_back(acc_sh, out_hbm, c, s)


assert NF == 125


@functools.partial(
    pl.kernel,
    out_type=jax.ShapeDtypeStruct((NC, N, D), jnp.float32),
    mesh=_mesh,
    scratch_types=[
        pltpu.VMEM((EPW,), jnp.int32),
        [pltpu.VMEM((CH,), jnp.int32) for _ in range(3)],
        [pltpu.VMEM((CH, D), jnp.float32) for _ in range(3)],
        pltpu.VMEM_SHARED((N, D), jnp.float32),
        [pltpu.SemaphoreType.DMA for _ in range(3)],
        [pltpu.SemaphoreType.DMA for _ in range(3)],
    ],
)
def _hop_kernel(g_hbm, src_hbm, dst_hbm, zeros_hbm, out_hbm,
                sidx_v, didx, rows, acc_sh, sg, si):
    c = lax.axis_index("c")
    s = lax.axis_index("s")
    wid = s * NC + c
    base = wid * EPW

    def issue_fetch(k, b):
        pltpu.async_copy(dst_hbm.at[pl.ds(base + k * CH, CH)], didx[b], si[b])
        pltpu.async_copy(g_hbm.at[sidx_v.at[pl.ds(k * CH, CH)]], rows[b], sg[b])

    def wait_fetch(k, b):
        pltpu.make_async_copy(dst_hbm.at[pl.ds(base + k * CH, CH)],
                              didx[b], si[b]).wait()
        pltpu.make_async_copy(g_hbm.at[sidx_v.at[pl.ds(k * CH, CH)]],
                              rows[b], sg[b]).wait()

    pltpu.sync_copy(src_hbm.at[pl.ds(base, EPW)], sidx_v)
    issue_fetch(0, 0)
    issue_fetch(1, 1)
    issue_fetch(2, 2)

    _zero_acc(zeros_hbm, acc_sh, s)
    plsc.subcore_barrier()

    def body(g, carry):
        v0 = 3 * g
        for u in range(3):
            v = v0 + u
            b = u
            wait_fetch(v, b)
            pltpu.sync_copy(rows[b], acc_sh.at[didx[b]], add=True)
            issue_fetch(v + 3, b)
        return carry

    lax.fori_loop(0, 40, body, 0)

    for v in (120, 121):
        b = v % 3
        wait_fetch(v, b)
        pltpu.sync_copy(rows[b], acc_sh.at[didx[b]], add=True)
        issue_fetch(v + 3, b)
    for v in (122, 123, 124):
        b = v % 3
        wait_fetch(v, b)
        pltpu.sync_copy(rows[b], acc_sh.at[didx[b]], add=True)

    plsc.subcore_barrier()
    _write_back(acc_sh, out_hbm, c, s)


_BR = 2000


def _norm_body(dacc_ref, feat_ref, g_ref, norm_ref):
    deg = dacc_ref[0, :, 0:1] + dacc_ref[1, :, 0:1]
    deg = jnp.maximum(deg, 1.0)
    nrm = lax.rsqrt(deg)
    norm_ref[...] = nrm
    g_ref[...] = feat_ref[...] * nrm


_norm_call = pl.pallas_call(
    _norm_body,
    grid=(N // _BR,),
    in_specs=[
        pl.BlockSpec((NC, _BR, D), lambda i: (0, i, 0)),
        pl.BlockSpec((_BR, D), lambda i: (i, 0)),
    ],
    out_specs=[
        pl.BlockSpec((_BR, D), lambda i: (i, 0)),
        pl.BlockSpec((_BR, 1), lambda i: (i, 0)),
    ],
    out_shape=[
        jax.ShapeDtypeStruct((N, D), jnp.float32),
        jax.ShapeDtypeStruct((N, 1), jnp.float32),
    ],
)


def _mid_body(p_ref, norm_ref, g_ref):
    nrm = norm_ref[...]
    g_ref[...] = (p_ref[0] + p_ref[1]) * (nrm * nrm)


_mid_call = pl.pallas_call(
    _mid_body,
    grid=(N // _BR,),
    in_specs=[
        pl.BlockSpec((NC, _BR, D), lambda i: (0, i, 0)),
        pl.BlockSpec((_BR, 1), lambda i: (i, 0)),
    ],
    out_specs=pl.BlockSpec((_BR, D), lambda i: (i, 0)),
    out_shape=jax.ShapeDtypeStruct((N, D), jnp.float32),
)


def _fin_body(q_ref, norm_ref, wt_ref, b_ref, out_ref):
    h = (q_ref[0] + q_ref[1]) * norm_ref[...]
    out_ref[...] = (
        jnp.dot(h, wt_ref[...], preferred_element_type=jnp.float32)
        + b_ref[...]
    )


_fin_call = pl.pallas_call(
    _fin_body,
    grid=(N // _BR,),
    in_specs=[
        pl.BlockSpec((NC, _BR, D), lambda i: (0, i, 0)),
        pl.BlockSpec((_BR, 1), lambda i: (i, 0)),
        pl.BlockSpec((D, D), lambda i: (0, 0)),
        pl.BlockSpec((1, D), lambda i: (0, 0)),
    ],
    out_specs=pl.BlockSpec((_BR, D), lambda i: (i, 0)),
    out_shape=jax.ShapeDtypeStruct((N, D), jnp.float32),
)


def kernel(feat, edge_index, W, b):
    ei = edge_index.astype(jnp.int32)
    src = ei[0]
    dst = ei[1]
    zeros = jnp.zeros((N, D), jnp.float32)
    ones = jnp.ones((DCH, D), jnp.float32)

    dacc = _deg_kernel(dst, zeros, ones)
    g1, norm = _norm_call(dacc, feat)
    p = _hop_kernel(g1, src, dst, zeros)
    g2 = _mid_call(p, norm)
    q = _hop_kernel(g2, src, dst, zeros)
    out = _fin_call(q, norm, W.T.astype(jnp.float32), b.reshape(1, D))
    return out


# speedup vs baseline: 1.1475x; 1.0001x over previous
import functools

import jax
import jax.numpy as jnp
from jax import lax
from jax.experimental import pallas as pl
from jax.experimental.pallas import tpu as pltpu
from jax.experimental.pallas import tpu_sc as plsc

N = 10000
E = 320000
D = 128
NC = 2
NS = 16
NW = NC * NS
EPW = E // NW

CH = 80
NF = EPW // CH

DCH = 128
DNF = EPW // DCH
DTEDGE = EPW - DNF * DCH
DTOFF = DNF * DCH

RPT = 624
TAIL = N - NS * RPT
TBASE = NS * RPT

_mesh = plsc.VectorSubcoreMesh(core_axis_name="c", subcore_axis_name="s")


def _zero_acc(zeros_hbm, acc_sh, s):
    pltpu.sync_copy(zeros_hbm.at[pl.ds(s * RPT, RPT)],
                    acc_sh.at[pl.ds(s * RPT, RPT)])

    @pl.when(s == 0)
    def _():
        pltpu.sync_copy(zeros_hbm.at[pl.ds(TBASE, TAIL)],
                        acc_sh.at[pl.ds(TBASE, TAIL)])


def _write_back(acc_sh, out_hbm, c, s):
    pltpu.sync_copy(acc_sh.at[pl.ds(s * RPT, RPT)],
                    out_hbm.at[c, pl.ds(s * RPT, RPT)])

    @pl.when(s == 0)
    def _():
        pltpu.sync_copy(acc_sh.at[pl.ds(TBASE, TAIL)],
                        out_hbm.at[c, pl.ds(TBASE, TAIL)])


assert DNF == 78


@functools.partial(
    pl.kernel,
    out_type=jax.ShapeDtypeStruct((NC, N, D), jnp.float32),
    mesh=_mesh,
    scratch_types=[
        [pltpu.VMEM((DCH,), jnp.int32) for _ in range(4)],
        pltpu.VMEM((DTEDGE,), jnp.int32),
        pltpu.VMEM((DCH, D), jnp.float32),
        pltpu.VMEM_SHARED((N, D), jnp.float32),
        [pltpu.SemaphoreType.DMA for _ in range(4)],
        [pltpu.SemaphoreType.DMA for _ in range(4)],
        pltpu.SemaphoreType.DMA,
    ],
)
def _deg_kernel(dst_hbm, zeros_hbm, ones_hbm, out_hbm,
                didx, didxt, ones_v, acc_sh, si, ss, sit):
    c = lax.axis_index("c")
    s = lax.axis_index("s")
    wid = s * NC + c
    base = wid * EPW

    def issue_fetch(k, b):
        pltpu.async_copy(dst_hbm.at[pl.ds(base + k * DCH, DCH)],
                         didx[b], si[b])

    def wait_fetch(k, b):
        pltpu.make_async_copy(dst_hbm.at[pl.ds(base + k * DCH, DCH)],
                              didx[b], si[b]).wait()

    def issue_scatter(b):
        pltpu.async_copy(ones_v, acc_sh.at[didx[b]], ss[b], add=True)

    def wait_scatter(b):
        pltpu.make_async_copy(ones_v, acc_sh.at[didx[b]], ss[b]).wait()

    issue_fetch(0, 0)
    issue_fetch(1, 1)
    pltpu.async_copy(dst_hbm.at[pl.ds(base + DTOFF, DTEDGE)], didxt, sit)
    pltpu.sync_copy(ones_hbm, ones_v)
    _zero_acc(zeros_hbm, acc_sh, s)
    plsc.subcore_barrier()

    for v in (0, 1):
        wait_fetch(v, v)
        issue_scatter(v)
        issue_fetch(v + 2, v + 2)
    for v in (2, 3):
        wait_fetch(v, v)
        issue_scatter(v)
        wait_scatter(v - 2)
        issue_fetch(v + 2, v - 2)

    def body(g, carry):
        v0 = 4 * g
        for u in range(4):
            v = v0 + u
            b = u % 4
            b2 = (u + 2) % 4
            wait_fetch(v, b)
            issue_scatter(b)
            wait_scatter(b2)
            issue_fetch(v + 2, b2)
        return carry

    lax.fori_loop(1, 19, body, 0)

    for v in (76, 77):
        wait_fetch(v, v % 4)
        issue_scatter(v % 4)

    pltpu.make_async_copy(dst_hbm.at[pl.ds(base + DTOFF, DTEDGE)],
                          didxt, sit).wait()
    pltpu.sync_copy(ones_v.at[pl.ds(0, DTEDGE)], acc_sh.at[didxt], add=True)

    for b in (2, 3, 0, 1):
        wait_scatter(b)

    plsc.subcore_barrier()
    _write_back(acc_sh, out_hbm, c, s)


assert NF == 125


@functools.partial(
    pl.kernel,
    out_type=jax.ShapeDtypeStruct((NC, N, D), jnp.float32),
    mesh=_mesh,
    scratch_types=[
        pltpu.VMEM((EPW,), jnp.int32),
        [pltpu.VMEM((CH,), jnp.int32) for _ in range(3)],
        [pltpu.VMEM((CH, D), jnp.float32) for _ in range(3)],
        pltpu.VMEM_SHARED((N, D), jnp.float32),
        [pltpu.SemaphoreType.DMA for _ in range(3)],
        [pltpu.SemaphoreType.DMA for _ in range(3)],
    ],
)
def _hop_kernel(g_hbm, src_hbm, dst_hbm, zeros_hbm, out_hbm,
                sidx_v, didx, rows, acc_sh, sg, si):
    c = lax.axis_index("c")
    s = lax.axis_index("s")
    wid = s * NC + c
    base = wid * EPW

    def issue_fetch(k, b):
        pltpu.async_copy(dst_hbm.at[pl.ds(base + k * CH, CH)], didx[b], si[b])
        pltpu.async_copy(g_hbm.at[sidx_v.at[pl.ds(k * CH, CH)]], rows[b], sg[b])

    def wait_fetch(k, b):
        pltpu.make_async_copy(dst_hbm.at[pl.ds(base + k * CH, CH)],
                              didx[b], si[b]).wait()
        pltpu.make_async_copy(g_hbm.at[sidx_v.at[pl.ds(k * CH, CH)]],
                              rows[b], sg[b]).wait()

    pltpu.sync_copy(src_hbm.at[pl.ds(base, EPW)], sidx_v)
    issue_fetch(0, 0)
    issue_fetch(1, 1)
    issue_fetch(2, 2)

    _zero_acc(zeros_hbm, acc_sh, s)
    plsc.subcore_barrier()

    def body(g, carry):
        v0 = 3 * g
        for u in range(3):
            v = v0 + u
            b = u
            wait_fetch(v, b)
            pltpu.sync_copy(rows[b], acc_sh.at[didx[b]], add=True)
            issue_fetch(v + 3, b)
        return carry

    lax.fori_loop(0, 40, body, 0)

    for v in (120, 121):
        b = v % 3
        wait_fetch(v, b)
        pltpu.sync_copy(rows[b], acc_sh.at[didx[b]], add=True)
        issue_fetch(v + 3, b)
    for v in (122, 123, 124):
        b = v % 3
        wait_fetch(v, b)
        pltpu.sync_copy(rows[b], acc_sh.at[didx[b]], add=True)

    plsc.subcore_barrier()
    _write_back(acc_sh, out_hbm, c, s)


_BR = 2000


def _norm_body(dacc_ref, feat_ref, g_ref, norm_ref):
    deg = dacc_ref[0, :, 0:1] + dacc_ref[1, :, 0:1]
    deg = jnp.maximum(deg, 1.0)
    nrm = lax.rsqrt(deg)
    norm_ref[...] = nrm
    g_ref[...] = feat_ref[...] * nrm


_norm_call = pl.pallas_call(
    _norm_body,
    grid=(N // _BR,),
    in_specs=[
        pl.BlockSpec((NC, _BR, D), lambda i: (0, i, 0)),
        pl.BlockSpec((_BR, D), lambda i: (i, 0)),
    ],
    out_specs=[
        pl.BlockSpec((_BR, D), lambda i: (i, 0)),
        pl.BlockSpec((_BR, 1), lambda i: (i, 0)),
    ],
    out_shape=[
        jax.ShapeDtypeStruct((N, D), jnp.float32),
        jax.ShapeDtypeStruct((N, 1), jnp.float32),
    ],
)


def _mid_body(p_ref, norm_ref, g_ref):
    nrm = norm_ref[...]
    g_ref[...] = (p_ref[0] + p_ref[1]) * (nrm * nrm)


_mid_call = pl.pallas_call(
    _mid_body,
    grid=(N // _BR,),
    in_specs=[
        pl.BlockSpec((NC, _BR, D), lambda i: (0, i, 0)),
        pl.BlockSpec((_BR, 1), lambda i: (i, 0)),
    ],
    out_specs=pl.BlockSpec((_BR, D), lambda i: (i, 0)),
    out_shape=jax.ShapeDtypeStruct((N, D), jnp.float32),
)


def _fin_body(q_ref, norm_ref, wt_ref, b_ref, out_ref):
    h = (q_ref[0] + q_ref[1]) * norm_ref[...]
    out_ref[...] = (
        jnp.dot(h, wt_ref[...], preferred_element_type=jnp.float32)
        + b_ref[...]
    )


_fin_call = pl.pallas_call(
    _fin_body,
    grid=(N // _BR,),
    in_specs=[
        pl.BlockSpec((NC, _BR, D), lambda i: (0, i, 0)),
        pl.BlockSpec((_BR, 1), lambda i: (i, 0)),
        pl.BlockSpec((D, D), lambda i: (0, 0)),
        pl.BlockSpec((1, D), lambda i: (0, 0)),
    ],
    out_specs=pl.BlockSpec((_BR, D), lambda i: (i, 0)),
    out_shape=jax.ShapeDtypeStruct((N, D), jnp.float32),
)


def kernel(feat, edge_index, W, b):
    ei = edge_index.astype(jnp.int32)
    src = ei[0]
    dst = ei[1]
    zeros = jnp.zeros((N, D), jnp.float32)
    ones = jnp.ones((DCH, D), jnp.float32)

    dacc = _deg_kernel(dst, zeros, ones)
    g1, norm = _norm_call(dacc, feat)
    p = _hop_kernel(g1, src, dst, zeros)
    g2 = _mid_call(p, norm)
    q = _hop_kernel(g2, src, dst, zeros)
    out = _fin_call(q, norm, W.T.astype(jnp.float32), b.reshape(1, D))
    return out
